# baseline (speedup 1.0000x reference)
import jax
import jax.numpy as jnp
from jax import lax
from jax.experimental import pallas as pl
from jax.experimental.pallas import tpu as pltpu
from jax.experimental.pallas import tpu_sc as plsc

LANES = 16
NC, NS = 2, 16
NW = NC * NS

B, V, L, D = 8, 8, 4096, 128
SEGS = B * V
SEGS_PER_W = SEGS // NW

L_SC = 1536
TC_B = L - L_SC

CHUNK = 384
NCHUNK = L_SC // CHUNK
ROW_UNROLL = 4
DV = D // LANES

TC_RING = 4


def _sc_body(x_hbm, out_hbm, buf, outv, sem0, sem1):
    wid = lax.axis_index("s") * NC + lax.axis_index("c")
    base_seg = wid * SEGS_PER_W
    sems = (sem0, sem1)

    for s in range(SEGS_PER_W):
        seg = base_seg + s

        def copy(g, k):
            return pltpu.make_async_copy(
                x_hbm.at[seg, pl.ds(g * CHUNK, CHUNK)],
                buf.at[pl.ds(k * CHUNK, CHUNK)],
                sems[k],
            )

        copy(0, 0).start()
        copy(1, 1).start()

        def pair_body(p, acc):
            for k in range(2):
                g = 2 * p + k
                copy(g, k).wait()

                def row_body(i, a, k=k):
                    r = k * CHUNK + i * ROW_UNROLL
                    out = list(a)
                    for u in range(ROW_UNROLL):
                        for j in range(DV):
                            out[j] = out[j] + buf[
                                r + u, pl.ds(j * LANES, LANES)
                            ]
                    return tuple(out)

                acc = lax.fori_loop(0, CHUNK // ROW_UNROLL, row_body, acc)

                @pl.when(g + 2 < NCHUNK)
                def _():
                    copy(g + 2, k).start()

            return acc

        acc = tuple(jnp.zeros((LANES,), jnp.float32) for _ in range(DV))
        acc = lax.fori_loop(0, NCHUNK // 2, pair_body, acc)
        for j in range(DV):
            outv[s, pl.ds(j * LANES, LANES)] = acc[j] * (1.0 / float(D))

    pltpu.sync_copy(outv, out_hbm.at[pl.ds(base_seg, SEGS_PER_W)])


def _tc_body(x_hbm, o_ref, bufs, sems):
    def copy(seg, slot):
        return pltpu.make_async_copy(
            x_hbm.at[seg, pl.ds(L_SC, TC_B)], bufs.at[slot], sems.at[slot]
        )

    for k in range(TC_RING):
        copy(k, k).start()

    def body(p, _):
        for k in range(TC_RING):
            seg = TC_RING * p + k
            copy(seg, k).wait()
            acc = jnp.sum(bufs[k], axis=0, keepdims=True)
            o_ref[pl.ds(seg, 1), :] = acc * (1.0 / float(D))

            @pl.when(seg + TC_RING < SEGS)
            def _():
                copy(seg + TC_RING, k).start()

        return 0

    lax.fori_loop(0, SEGS // TC_RING, body, 0)


@jax.jit
def kernel(X):
    xf = X.reshape(SEGS, L, D)

    sc_part = pl.kernel(
        _sc_body,
        out_type=jax.ShapeDtypeStruct((SEGS, D), jnp.float32),
        mesh=plsc.VectorSubcoreMesh(core_axis_name="c", subcore_axis_name="s"),
        scratch_types=[
            pltpu.VMEM((2 * CHUNK, D), jnp.float32),
            pltpu.VMEM((SEGS_PER_W, D), jnp.float32),
            pltpu.SemaphoreType.DMA,
            pltpu.SemaphoreType.DMA,
        ],
    )(xf)

    tc_part = pl.pallas_call(
        _tc_body,
        in_specs=[pl.BlockSpec(memory_space=pl.ANY)],
        out_specs=pl.BlockSpec(memory_space=pltpu.VMEM),
        out_shape=jax.ShapeDtypeStruct((SEGS, D), jnp.float32),
        scratch_shapes=[
            pltpu.VMEM((TC_RING, TC_B, D), jnp.float32),
            pltpu.SemaphoreType.DMA((TC_RING,)),
        ],
    )(xf)

    ret = sc_part + tc_part
    ret = jnp.where(jnp.isnan(ret), jnp.zeros_like(ret), ret)
    return ret.reshape(B, V, D)


# speedup vs baseline: 1.0049x; 1.0049x over previous
import jax
import jax.numpy as jnp
from jax import lax
from jax.experimental import pallas as pl
from jax.experimental.pallas import tpu as pltpu
from jax.experimental.pallas import tpu_sc as plsc

LANES = 16
NC, NS = 2, 16
NW = NC * NS

B, V, L, D = 8, 8, 4096, 128
SEGS = B * V
SEGS_PER_W = SEGS // NW

L_SC = 1280
TC_B = L - L_SC

CHUNK = 320
NCHUNK = L_SC // CHUNK
ROW_UNROLL = 4
DV = D // LANES

TC_RING = 4


def _sc_body(x_hbm, out_hbm, buf, outv, sem0, sem1):
    wid = lax.axis_index("s") * NC + lax.axis_index("c")
    base_seg = wid * SEGS_PER_W
    sems = (sem0, sem1)

    for s in range(SEGS_PER_W):
        seg = base_seg + s

        def copy(g, k):
            return pltpu.make_async_copy(
                x_hbm.at[seg, pl.ds(g * CHUNK, CHUNK)],
                buf.at[pl.ds(k * CHUNK, CHUNK)],
                sems[k],
            )

        copy(0, 0).start()
        copy(1, 1).start()

        def pair_body(p, acc):
            for k in range(2):
                g = 2 * p + k
                copy(g, k).wait()

                def row_body(i, a, k=k):
                    r = k * CHUNK + i * ROW_UNROLL
                    out = list(a)
                    for u in range(ROW_UNROLL):
                        for j in range(DV):
                            out[j] = out[j] + buf[
                                r + u, pl.ds(j * LANES, LANES)
                            ]
                    return tuple(out)

                acc = lax.fori_loop(0, CHUNK // ROW_UNROLL, row_body, acc)

                @pl.when(g + 2 < NCHUNK)
                def _():
                    copy(g + 2, k).start()

            return acc

        acc = tuple(jnp.zeros((LANES,), jnp.float32) for _ in range(DV))
        acc = lax.fori_loop(0, NCHUNK // 2, pair_body, acc)
        for j in range(DV):
            outv[s, pl.ds(j * LANES, LANES)] = acc[j] * (1.0 / float(D))

    pltpu.sync_copy(outv, out_hbm.at[pl.ds(base_seg, SEGS_PER_W)])


def _tc_body(x_hbm, o_ref, bufs, sems):
    def copy(seg, slot):
        return pltpu.make_async_copy(
            x_hbm.at[seg, pl.ds(L_SC, TC_B)], bufs.at[slot], sems.at[slot]
        )

    for k in range(TC_RING):
        copy(k, k).start()

    def body(p, _):
        for k in range(TC_RING):
            seg = TC_RING * p + k
            copy(seg, k).wait()
            acc = jnp.sum(bufs[k], axis=0, keepdims=True)
            o_ref[pl.ds(seg, 1), :] = acc * (1.0 / float(D))

            @pl.when(seg + TC_RING < SEGS)
            def _():
                copy(seg + TC_RING, k).start()

        return 0

    lax.fori_loop(0, SEGS // TC_RING, body, 0)


@jax.jit
def kernel(X):
    xf = X.reshape(SEGS, L, D)

    sc_part = pl.kernel(
        _sc_body,
        out_type=jax.ShapeDtypeStruct((SEGS, D), jnp.float32),
        mesh=plsc.VectorSubcoreMesh(core_axis_name="c", subcore_axis_name="s"),
        scratch_types=[
            pltpu.VMEM((2 * CHUNK, D), jnp.float32),
            pltpu.VMEM((SEGS_PER_W, D), jnp.float32),
            pltpu.SemaphoreType.DMA,
            pltpu.SemaphoreType.DMA,
        ],
    )(xf)

    tc_part = pl.pallas_call(
        _tc_body,
        in_specs=[pl.BlockSpec(memory_space=pl.ANY)],
        out_specs=pl.BlockSpec(memory_space=pltpu.VMEM),
        out_shape=jax.ShapeDtypeStruct((SEGS, D), jnp.float32),
        scratch_shapes=[
            pltpu.VMEM((TC_RING, TC_B, D), jnp.float32),
            pltpu.SemaphoreType.DMA((TC_RING,)),
        ],
    )(xf)

    ret = sc_part + tc_part
    ret = jnp.where(jnp.isnan(ret), jnp.zeros_like(ret), ret)
    return ret.reshape(B, V, D)
